# baseline (speedup 1.0000x reference)
import functools

import jax
import jax.numpy as jnp
from jax import lax
from jax.experimental import pallas as pl
from jax.experimental.pallas import tpu as pltpu
from jax.experimental.pallas import tpu_sc as plsc

NC = 2
NS = 16
NW = NC * NS
L = 16

DIM = 64
G = 16


def _make_sc_kernel(batch: int):
    b_per_w = batch // NW
    n_groups = b_per_w // G

    mesh = plsc.VectorSubcoreMesh(core_axis_name="c", subcore_axis_name="s")

    @functools.partial(
        pl.kernel,
        out_type=jax.ShapeDtypeStruct((NW, b_per_w), jnp.float32),
        mesh=mesh,
        compiler_params=pltpu.CompilerParams(
            needs_layout_passes=False, use_tc_tiling_on_sc=True),
        scratch_types=[
            pltpu.VMEM((b_per_w,), jnp.int32),
            pltpu.VMEM((b_per_w,), jnp.int32),
            pltpu.VMEM((3, G // 8, 8, 8, 128), jnp.float32),
            pltpu.VMEM((3, G // 8, 8, 8, 128), jnp.float32),
            pltpu.VMEM((b_per_w,), jnp.float32),
            pltpu.SemaphoreType.DMA((3, 2)),
            pltpu.SemaphoreType.DMA((3, 2)),
        ],
    )
    def sc_kernel(u_hbm, i_hbm, ut_hbm, it_hbm, out_hbm,
                  uidx_v, iidx_v, ublk_v, iblk_v, scores_v, usem, isem):
        wid = lax.axis_index("s") * NC + lax.axis_index("c")

        pltpu.sync_copy(u_hbm.at[wid], uidx_v)
        pltpu.sync_copy(i_hbm.at[wid], iidx_v)

        iota = lax.iota(jnp.int32, L)
        lane_mask = iota == (L - 1)
        a_idx = [jnp.asarray(((16 * k + jnp.arange(L)) // 8).astype(jnp.int32))
                 for k in range(4)]
        s_idx = [jnp.asarray(((16 * k + jnp.arange(L)) % 8).astype(jnp.int32))
                 for k in range(4)]

        def issue_group(g, parity):
            base = pl.multiple_of(g * G, G)
            uvec = uidx_v[pl.ds(base, G)]
            ivec = iidx_v[pl.ds(base, G)]
            for j in range(G):
                off_u = pl.multiple_of(uvec[j] & ~(L - 1), L)
                off_i = pl.multiple_of(ivec[j] & ~(L - 1), L)
                slot = pl.ds(L * (j % 8), L)
                pltpu.async_copy(
                    ut_hbm.at[:, :, pl.ds(off_u, L)],
                    ublk_v.at[parity, j // 8, :, :, slot],
                    usem.at[parity, j % 2])
                pltpu.async_copy(
                    it_hbm.at[:, :, pl.ds(off_i, L)],
                    iblk_v.at[parity, j // 8, :, :, slot],
                    isem.at[parity, j % 2])

        def compute_group(g, parity):
            for q in range(2):
                pltpu.make_async_copy(
                    ut_hbm.at[:, :, pl.ds(0, 128)],
                    ublk_v.at[parity, q], usem.at[parity, q]).wait()
                pltpu.make_async_copy(
                    it_hbm.at[:, :, pl.ds(0, 128)],
                    iblk_v.at[parity, q], isem.at[parity, q]).wait()

            base = pl.multiple_of(g * G, G)
            uvec = uidx_v[pl.ds(base, G)]
            ivec = iidx_v[pl.ds(base, G)]
            cu_all = uvec & (L - 1)
            ci_all = ivec & (L - 1)
            for j in range(G):
                cu = lax.broadcast(cu_all[j] + L * (j % 8), (L,))
                ci = lax.broadcast(ci_all[j] + L * (j % 8), (L,))
                ublk = ublk_v.at[parity, j // 8]
                iblk = iblk_v.at[parity, j // 8]
                prods = []
                for k in range(4):
                    eu = plsc.load_gather(ublk, [a_idx[k], s_idx[k], cu])
                    ei = plsc.load_gather(iblk, [a_idx[k], s_idx[k], ci])
                    prods.append(eu * ei)
                acc = (prods[0] + prods[1]) + (prods[2] + prods[3])
                total = plsc.cumsum(acc)
                pos = lax.broadcast(g * G + j, (L,))
                plsc.store_scatter(scores_v, [pos], total, mask=lane_mask)

        def body(g, carry):
            @pl.when(g < n_groups)
            def _():
                issue_group(g, lax.rem(g, 3))

            @pl.when(g >= 2)
            def _():
                compute_group(g - 2, lax.rem(g - 2, 3))

            return carry

        lax.fori_loop(0, n_groups + 2, body, 0, unroll=False)

        pltpu.sync_copy(scores_v, out_hbm.at[wid])

    return sc_kernel


N_TC = 2048
GT = 16


def _make_tc_kernel(n_tc: int):
    n_groups = n_tc // GT

    def tc_fn(u_sref, i_sref, ut_hbm, it_hbm, out_hbm,
              ublk_v, iblk_v, out_v, usem, isem, osem):

        def issue_group(g, parity):
            for j in range(GT):
                off_u = pl.multiple_of(
                    (u_sref[g * GT + j] >> 7) << 7, 128)
                off_i = pl.multiple_of(
                    (i_sref[g * GT + j] >> 7) << 7, 128)
                pltpu.async_copy(
                    ut_hbm.at[:, :, pl.ds(off_u, 128)],
                    ublk_v.at[parity, j], usem)
                pltpu.async_copy(
                    it_hbm.at[:, :, pl.ds(off_i, 128)],
                    iblk_v.at[parity, j], isem)

        def compute_group(g, parity):
            for j in range(GT):
                pltpu.make_async_copy(
                    ut_hbm.at[:, :, pl.ds(0, 128)],
                    ublk_v.at[parity, j], usem).wait()
                pltpu.make_async_copy(
                    it_hbm.at[:, :, pl.ds(0, 128)],
                    iblk_v.at[parity, j], isem).wait()
            for j in range(GT):
                cu = u_sref[g * GT + j] & 127
                ci = i_sref[g * GT + j] & 127
                ub = ublk_v[parity, j].reshape(DIM, 128)
                ib = iblk_v[parity, j].reshape(DIM, 128)
                ibr = pltpu.roll(ib, (cu - ci) & 127, 1)
                prod = ub * ibr
                red = jnp.sum(prod, axis=0, keepdims=True)
                out_v[pl.ds(g * GT + j, 1), :] = red

        def body(g, carry):
            @pl.when(g < n_groups)
            def _():
                issue_group(g, lax.rem(g, 3))

            @pl.when(g >= 2)
            def _():
                compute_group(g - 2, lax.rem(g - 2, 3))

            return carry

        lax.fori_loop(0, n_groups + 2, body, 0, unroll=False)
        pltpu.async_copy(out_v, out_hbm, osem).wait()

    grid_spec = pltpu.PrefetchScalarGridSpec(
        num_scalar_prefetch=2,
        grid=(1,),
        in_specs=[pl.BlockSpec(memory_space=pltpu.HBM),
                  pl.BlockSpec(memory_space=pltpu.HBM)],
        out_specs=pl.BlockSpec(memory_space=pltpu.HBM),
        scratch_shapes=[
            pltpu.VMEM((3, GT, 8, DIM // 8, 128), jnp.float32),
            pltpu.VMEM((3, GT, 8, DIM // 8, 128), jnp.float32),
            pltpu.VMEM((n_tc, 128), jnp.float32),
            pltpu.SemaphoreType.DMA,
            pltpu.SemaphoreType.DMA,
            pltpu.SemaphoreType.DMA,
        ],
    )
    return pl.pallas_call(
        tc_fn,
        grid_spec=grid_spec,
        out_shape=jax.ShapeDtypeStruct((n_tc, 128), jnp.float32),
    )


@jax.jit
def kernel(u, i, user_emb, item_emb):
    batch = u.shape[0]
    n_rows, dim = user_emb.shape
    ut3 = user_emb.T.reshape(8, dim // 8, n_rows)
    it3 = item_emb.T.reshape(8, dim // 8, n_rows)
    n_sc = batch - N_TC
    u_sc, u_tc = u[:n_sc], u[n_sc:]
    i_sc, i_tc = i[:n_sc], i[n_sc:]
    u_r = u_sc.reshape(NW, n_sc // NW)
    i_r = i_sc.reshape(NW, n_sc // NW)
    sc_scores = _make_sc_kernel(n_sc)(u_r, i_r, ut3, it3)
    tc_out = _make_tc_kernel(N_TC)(u_tc, i_tc, ut3, it3)
    tc_scores = jnp.take_along_axis(
        tc_out, (u_tc & 127)[:, None], axis=1)[:, 0]
    return jnp.concatenate([sc_scores.reshape(n_sc), tc_scores])


# speedup vs baseline: 1.4977x; 1.4977x over previous
import functools

import jax
import jax.numpy as jnp
from jax import lax
from jax.experimental import pallas as pl
from jax.experimental.pallas import tpu as pltpu
from jax.experimental.pallas import tpu_sc as plsc

NC = 2
NS = 16
NW = NC * NS
L = 16

DIM = 64
G = 16


def _make_sc_kernel(batch: int):
    b_per_w = batch // NW
    n_groups = b_per_w // G

    mesh = plsc.VectorSubcoreMesh(core_axis_name="c", subcore_axis_name="s")

    @functools.partial(
        pl.kernel,
        out_type=jax.ShapeDtypeStruct((NW, b_per_w), jnp.float32),
        mesh=mesh,
        compiler_params=pltpu.CompilerParams(
            needs_layout_passes=False, use_tc_tiling_on_sc=True),
        scratch_types=[
            pltpu.VMEM((b_per_w,), jnp.int32),
            pltpu.VMEM((b_per_w,), jnp.int32),
            pltpu.VMEM((3, G // 8, 8, 8, 128), jnp.float32),
            pltpu.VMEM((3, G // 8, 8, 8, 128), jnp.float32),
            pltpu.VMEM((b_per_w,), jnp.float32),
            pltpu.SemaphoreType.DMA((3, 2)),
            pltpu.SemaphoreType.DMA((3, 2)),
        ],
    )
    def sc_kernel(u_hbm, i_hbm, ut_hbm, it_hbm, out_hbm,
                  uidx_v, iidx_v, ublk_v, iblk_v, scores_v, usem, isem):
        wid = lax.axis_index("s") * NC + lax.axis_index("c")

        pltpu.sync_copy(u_hbm.at[wid], uidx_v)
        pltpu.sync_copy(i_hbm.at[wid], iidx_v)

        iota = lax.iota(jnp.int32, L)
        lane_mask = iota == (L - 1)
        a_idx = [jnp.asarray(((16 * k + jnp.arange(L)) // 8).astype(jnp.int32))
                 for k in range(4)]
        s_idx = [jnp.asarray(((16 * k + jnp.arange(L)) % 8).astype(jnp.int32))
                 for k in range(4)]

        def issue_group(g, parity):
            base = pl.multiple_of(g * G, G)
            uvec = uidx_v[pl.ds(base, G)]
            ivec = iidx_v[pl.ds(base, G)]
            for j in range(G):
                off_u = pl.multiple_of(uvec[j] & ~(L - 1), L)
                off_i = pl.multiple_of(ivec[j] & ~(L - 1), L)
                slot = pl.ds(L * (j % 8), L)
                pltpu.async_copy(
                    ut_hbm.at[:, :, pl.ds(off_u, L)],
                    ublk_v.at[parity, j // 8, :, :, slot],
                    usem.at[parity, j % 2])
                pltpu.async_copy(
                    it_hbm.at[:, :, pl.ds(off_i, L)],
                    iblk_v.at[parity, j // 8, :, :, slot],
                    isem.at[parity, j % 2])

        def compute_group(g, parity):
            for q in range(2):
                pltpu.make_async_copy(
                    ut_hbm.at[:, :, pl.ds(0, 128)],
                    ublk_v.at[parity, q], usem.at[parity, q]).wait()
                pltpu.make_async_copy(
                    it_hbm.at[:, :, pl.ds(0, 128)],
                    iblk_v.at[parity, q], isem.at[parity, q]).wait()

            base = pl.multiple_of(g * G, G)
            uvec = uidx_v[pl.ds(base, G)]
            ivec = iidx_v[pl.ds(base, G)]
            cu_all = uvec & (L - 1)
            ci_all = ivec & (L - 1)
            for j in range(G):
                cu = lax.broadcast(cu_all[j] + L * (j % 8), (L,))
                ci = lax.broadcast(ci_all[j] + L * (j % 8), (L,))
                ublk = ublk_v.at[parity, j // 8]
                iblk = iblk_v.at[parity, j // 8]
                prods = []
                for k in range(4):
                    eu = plsc.load_gather(ublk, [a_idx[k], s_idx[k], cu])
                    ei = plsc.load_gather(iblk, [a_idx[k], s_idx[k], ci])
                    prods.append(eu * ei)
                acc = (prods[0] + prods[1]) + (prods[2] + prods[3])
                total = plsc.cumsum(acc)
                pos = lax.broadcast(g * G + j, (L,))
                plsc.store_scatter(scores_v, [pos], total, mask=lane_mask)

        def body(g, carry):
            @pl.when(g < n_groups)
            def _():
                issue_group(g, lax.rem(g, 3))

            @pl.when(g >= 2)
            def _():
                compute_group(g - 2, lax.rem(g - 2, 3))

            return carry

        lax.fori_loop(0, n_groups + 2, body, 0, unroll=False)

        pltpu.sync_copy(scores_v, out_hbm.at[wid])

    return sc_kernel


N_TC = 2048
GT = 16


def _make_tc_kernel(n_tc: int):
    n_groups = n_tc // GT

    def tc_fn(u_sref, i_sref, ut_hbm, it_hbm, out_hbm,
              ublk_v, iblk_v, out_v, usem, isem, osem):

        def issue_group(g, parity):
            for j in range(GT):
                off_u = pl.multiple_of(
                    (u_sref[g * GT + j] >> 7) << 7, 128)
                off_i = pl.multiple_of(
                    (i_sref[g * GT + j] >> 7) << 7, 128)
                pltpu.async_copy(
                    ut_hbm.at[:, :, pl.ds(off_u, 128)],
                    ublk_v.at[parity, j], usem)
                pltpu.async_copy(
                    it_hbm.at[:, :, pl.ds(off_i, 128)],
                    iblk_v.at[parity, j], isem)

        def compute_group(g, parity):
            for j in range(GT):
                pltpu.make_async_copy(
                    ut_hbm.at[:, :, pl.ds(0, 128)],
                    ublk_v.at[parity, j], usem).wait()
                pltpu.make_async_copy(
                    it_hbm.at[:, :, pl.ds(0, 128)],
                    iblk_v.at[parity, j], isem).wait()
            for j in range(GT):
                cu = u_sref[g * GT + j] & 127
                ci = i_sref[g * GT + j] & 127
                ub = ublk_v[parity, j].reshape(DIM, 128)
                ib = iblk_v[parity, j].reshape(DIM, 128)
                ubr = pltpu.roll(ub, (128 - cu) & 127, 1)
                ibr = pltpu.roll(ib, (128 - ci) & 127, 1)
                prod = ubr * ibr
                red = jnp.sum(prod, axis=0, keepdims=True)
                out_v[pl.ds(g * GT + j, 1), :] = red

        def body(g, carry):
            @pl.when(g < n_groups)
            def _():
                issue_group(g, lax.rem(g, 2))

            @pl.when(g > 0)
            def _():
                compute_group(g - 1, lax.rem(g - 1, 2))

            return carry

        lax.fori_loop(0, n_groups + 1, body, 0, unroll=False)
        pltpu.async_copy(out_v, out_hbm, osem).wait()

    grid_spec = pltpu.PrefetchScalarGridSpec(
        num_scalar_prefetch=2,
        grid=(1,),
        in_specs=[pl.BlockSpec(memory_space=pltpu.HBM),
                  pl.BlockSpec(memory_space=pltpu.HBM)],
        out_specs=pl.BlockSpec(memory_space=pltpu.HBM),
        scratch_shapes=[
            pltpu.VMEM((2, GT, 8, DIM // 8, 128), jnp.float32),
            pltpu.VMEM((2, GT, 8, DIM // 8, 128), jnp.float32),
            pltpu.VMEM((n_tc, 128), jnp.float32),
            pltpu.SemaphoreType.DMA,
            pltpu.SemaphoreType.DMA,
            pltpu.SemaphoreType.DMA,
        ],
    )
    return pl.pallas_call(
        tc_fn,
        grid_spec=grid_spec,
        out_shape=jax.ShapeDtypeStruct((n_tc, 128), jnp.float32),
    )


@jax.jit
def kernel(u, i, user_emb, item_emb):
    batch = u.shape[0]
    n_rows, dim = user_emb.shape
    ut3 = user_emb.T.reshape(8, dim // 8, n_rows)
    it3 = item_emb.T.reshape(8, dim // 8, n_rows)
    n_sc = batch - N_TC
    u_sc, u_tc = u[:n_sc], u[n_sc:]
    i_sc, i_tc = i[:n_sc], i[n_sc:]
    u_r = u_sc.reshape(NW, n_sc // NW)
    i_r = i_sc.reshape(NW, n_sc // NW)
    sc_scores = _make_sc_kernel(n_sc)(u_r, i_r, ut3, it3)
    tc_out = _make_tc_kernel(N_TC)(u_tc, i_tc, ut3, it3)
    tc_scores = tc_out[:, 0]
    return jnp.concatenate([sc_scores.reshape(n_sc), tc_scores])
